# baseline (speedup 1.0000x reference)
import functools

import numpy as _np

import jax
import jax.numpy as jnp
from jax import lax
from jax.experimental import pallas as pl
from jax.experimental.pallas import tpu as pltpu
from jax.experimental.pallas import tpu_sc as plsc

_N = 10000
_D = 128
_E = 320000
_NREL = 200
_NC = 2
_NS = 16
_NW = _NC * _NS
_EPW = _E // _NW
_B = 80
_NCH = _EPW // _B
_NPAD = 10112
_RPS = _NPAD // _NS
_ZR = 8

_mesh = plsc.VectorSubcoreMesh(core_axis_name="c", subcore_axis_name="s")

_GDN = lax.GatherDimensionNumbers(
    offset_dims=(), collapsed_slice_dims=(0,), start_index_map=(0,))


def _vgather(vec16, idx16):
    return lax.gather(vec16, idx16.reshape(16, 1), _GDN, (1,),
                      mode=lax.GatherScatterMode.PROMISE_IN_BOUNDS)


def _lane_splat(vec16, lane):
    return _vgather(vec16, jnp.full((16,), lane, jnp.int32))


_NAT = 13


def _alpha16(alpha_v, rel16):
    acc = jnp.zeros((16,), jnp.float32)
    for t in range(_NAT):
        at = alpha_v[pl.ds(t * 16, 16)]
        off = rel16 - (t * 16)
        m = (off >= 0) & (off < 16)
        g = _vgather(at, jnp.clip(off, 0, 15))
        acc = jnp.where(m, g, acc)
    return acc


@functools.partial(
    pl.kernel,
    out_type=jax.ShapeDtypeStruct((_NC, _NPAD, _D), jnp.float32),
    mesh=_mesh,
    compiler_params=pltpu.CompilerParams(use_tc_tiling_on_sc=False,
                                         needs_layout_passes=False),
    scratch_types=[
        pltpu.VMEM((2, _B), jnp.int32),
        pltpu.VMEM((2, _B), jnp.int32),
        pltpu.VMEM((2, _B // 2), jnp.int32),
        pltpu.VMEM((2, _B // 2), jnp.int32),
        pltpu.VMEM((_NAT * 16,), jnp.float32),
        pltpu.VMEM((_B, _D), jnp.bfloat16),
        pltpu.VMEM((_B, _D), jnp.bfloat16),
        pltpu.VMEM((_B // 2, _D), jnp.float32),
        pltpu.VMEM((_B // 2, _D), jnp.float32),
        pltpu.VMEM((_ZR, _D), jnp.float32),
        pltpu.VMEM_SHARED((_NPAD, _D), jnp.float32),
        pltpu.SemaphoreType.DMA,
        pltpu.SemaphoreType.DMA,
        pltpu.SemaphoreType.DMA,
        pltpu.SemaphoreType.DMA,
        pltpu.SemaphoreType.DMA,
        pltpu.SemaphoreType.DMA,
        pltpu.SemaphoreType.DMA,
        pltpu.SemaphoreType.DMA,
    ],
)
def _sc_agg(h_hbm, sr_hbm, dstr_hbm, alpha_hbm, out_hbm,
            pk0, pk1, dc0, dc1, alpha_v, rw0, rw1, sbA, sbB, zbuf_v, agg_sh,
            sp0, sp1, sd0, sd1, sr0, sr1, scA, scB):
    cid = lax.axis_index("c")
    sid = lax.axis_index("s")
    wid = cid * _NS + sid
    pks, dcs, rws = [pk0, pk1], [dc0, dc1], [rw0, rw1]
    sbs, scs = [sbA, sbB], [scA, scB]
    sps, sds, srs = [sp0, sp1], [sd0, sd1], [sr0, sr1]

    pltpu.sync_copy(alpha_hbm, alpha_v)

    zv = jnp.zeros((16,), jnp.float32)
    for r in range(_ZR):
        for c in range(_D // 16):
            zbuf_v[r, pl.ds(c * 16, 16)] = zv

    def _zcp(k, carry):
        pltpu.sync_copy(zbuf_v, agg_sh.at[pl.ds(sid * _RPS + k * _ZR, _ZR)])
        return carry

    lax.fori_loop(0, _RPS // _ZR, _zcp, 0)
    plsc.subcore_barrier()

    def _issue_pk(j, b):
        pltpu.async_copy(sr_hbm.at[wid, j], pks[b], sps[b])

    def _issue_dst(j, b):
        pltpu.async_copy(dstr_hbm.at[wid, j], dcs[b], sds[b])

    def _issue_gather(b):
        pltpu.async_copy(h_hbm.at[pks[b].at[0]], rws[b], srs[b])

    def _wait_pk(b):
        pltpu.make_async_copy(sr_hbm.at[wid, 0], pks[b], sps[b]).wait()

    def _wait_dst(b):
        pltpu.make_async_copy(dstr_hbm.at[wid, 0], dcs[b], sds[b]).wait()

    def _wait_gather(b):
        pltpu.make_async_copy(h_hbm.at[pks[b].at[0]], rws[b], srs[b]).wait()

    def _issue_scatter(b, h):
        pltpu.async_copy(sbs[h], agg_sh.at[dcs[b].at[h]], scs[h], add=True)

    def _wait_scatter(h):
        pltpu.make_async_copy(sbs[h], agg_sh.at[dcs[0].at[h]], scs[h]).wait()

    def _lookup(b):
        return [_alpha16(alpha_v, pks[b][1, pl.ds(eb * 16, 16)])
                for eb in range(_B // 16)]

    def _scale_half(b, a16s, h):
        for el in range(_B // 2):
            e = h * (_B // 2) + el
            ae = _lane_splat(a16s[e // 16], e % 16)
            for g in range(_D // 32):
                v = rws[b][e, pl.ds(g * 32, 32)]
                pa, pb = plsc.unpack(v, format=plsc.PackFormat.INTERLEAVED)
                sbs[h][el, pl.ds(g * 32, 16)] = pa * ae
                sbs[h][el, pl.ds(g * 32 + 16, 16)] = pb * ae

    def _sub_iter(j, b):
        b1 = 1 - b
        _wait_pk(b1)
        _issue_gather(b1)
        a16s = _lookup(b)
        _issue_pk(jnp.minimum(j + 2, _NCH - 1), b)
        _wait_gather(b)
        _wait_dst(b)
        _wait_scatter(0)
        _scale_half(b, a16s, 0)
        _issue_scatter(b, 0)
        _wait_scatter(1)
        _issue_dst(jnp.minimum(j + 1, _NCH - 1), b1)
        _scale_half(b, a16s, 1)
        _issue_scatter(b, 1)

    _issue_pk(0, 0)
    _issue_pk(1, 1)
    _issue_dst(0, 0)
    _issue_dst(1, 1)
    _wait_pk(0)
    _issue_gather(0)
    _wait_pk(1)
    _issue_gather(1)
    a16s0 = _lookup(0)
    _issue_pk(2, 0)
    _wait_gather(0)
    _wait_dst(0)
    _scale_half(0, a16s0, 0)
    _issue_scatter(0, 0)
    _scale_half(0, a16s0, 1)
    _issue_scatter(0, 1)

    def _pair(i, carry):
        _sub_iter(2 * i + 1, 1)
        _sub_iter(2 * i + 2, 0)
        return carry

    lax.fori_loop(0, (_NCH - 1) // 2, _pair, 0)

    _wait_scatter(0)
    _wait_scatter(1)
    _wait_pk(0)
    _wait_dst(1)
    _wait_gather(1)

    plsc.subcore_barrier()

    sl = pl.ds(sid * _RPS, _RPS)
    pltpu.sync_copy(agg_sh.at[sl], out_hbm.at[cid].at[sl])


def _tc_body(agg_ref, h_ref, w_ref, b_ref, g_ref, be_ref, out_ref):
    x = agg_ref[0, :_N] + agg_ref[1, :_N] + h_ref[...]
    y = jnp.dot(x, w_ref[...], preferred_element_type=jnp.float32)
    y = y + b_ref[...]
    mu = jnp.mean(y, axis=0, keepdims=True)
    d = y - mu
    var = jnp.mean(d * d, axis=0, keepdims=True)
    out_ref[...] = jnp.tanh(d * lax.rsqrt(var + 1e-5) * g_ref[...] + be_ref[...])


_tc_layer = pl.pallas_call(
    _tc_body,
    out_shape=jax.ShapeDtypeStruct((_N, _D), jnp.float32),
)


_PERM = tuple(32 * g + 2 * k + p
              for g in range(_D // 32) for p in range(2) for k in range(16))
_PERM_ARR = _np.asarray(_PERM, dtype=_np.int32)


def kernel(entity_embed, edge, alpha0, W0, b0, gamma0, beta0,
           alpha1, W1, b1, gamma1, beta1):
    edge = edge.astype(jnp.int32)
    src = edge[:, 0].reshape(_NW, _NCH, 1, _B)
    rel = (edge[:, 1] % _NREL).reshape(_NW, _NCH, 1, _B)
    sr = jnp.concatenate([src, rel], axis=2)
    dst = edge[:, 2].reshape(_NW, _NCH, 2, _B // 2)
    apad = jnp.zeros((_NAT * 16 - _NREL,), jnp.float32)
    a0 = jnp.concatenate([alpha0, apad])
    a1 = jnp.concatenate([alpha1, apad])
    b0r, g0r, be0r = b0.reshape(1, _D), gamma0.reshape(1, _D), beta0.reshape(1, _D)
    b1r, g1r, be1r = b1.reshape(1, _D), gamma1.reshape(1, _D), beta1.reshape(1, _D)

    hb0 = entity_embed.astype(jnp.bfloat16)
    hp0 = jnp.take(entity_embed, _PERM_ARR, axis=1)
    agg = _sc_agg(hb0, sr, dst, a0)
    h1 = _tc_layer(agg, hp0, W0[_PERM_ARR, :], b0r, g0r, be0r)
    hb1 = h1.astype(jnp.bfloat16)
    hp1 = jnp.take(h1, _PERM_ARR, axis=1)
    agg2 = _sc_agg(hb1, sr, dst, a1)
    h2 = _tc_layer(agg2, hp1, W1[_PERM_ARR, :], b1r, g1r, be1r)
    return h2


# speedup vs baseline: 1.0072x; 1.0072x over previous
import functools

import numpy as _np

import jax
import jax.numpy as jnp
from jax import lax
from jax.experimental import pallas as pl
from jax.experimental.pallas import tpu as pltpu
from jax.experimental.pallas import tpu_sc as plsc

_N = 10000
_D = 128
_E = 320000
_NREL = 200
_NC = 2
_NS = 16
_NW = _NC * _NS
_EPW = _E // _NW
_B = 80
_NCH = _EPW // _B
_NPAD = 10112
_RPS = _NPAD // _NS
_ZR = 8

_mesh = plsc.VectorSubcoreMesh(core_axis_name="c", subcore_axis_name="s")

_GDN = lax.GatherDimensionNumbers(
    offset_dims=(), collapsed_slice_dims=(0,), start_index_map=(0,))


def _vgather(vec16, idx16):
    return lax.gather(vec16, idx16.reshape(16, 1), _GDN, (1,),
                      mode=lax.GatherScatterMode.PROMISE_IN_BOUNDS)


def _lane_splat(vec16, lane):
    return _vgather(vec16, jnp.full((16,), lane, jnp.int32))


_NAT = 13


def _alpha16(alpha_v, rel16):
    acc = jnp.zeros((16,), jnp.float32)
    for t in range(_NAT):
        at = alpha_v[pl.ds(t * 16, 16)]
        off = rel16 - (t * 16)
        m = (off >= 0) & (off < 16)
        g = _vgather(at, jnp.clip(off, 0, 15))
        acc = jnp.where(m, g, acc)
    return acc


@functools.partial(
    pl.kernel,
    out_type=jax.ShapeDtypeStruct((_NC, _NPAD, _D), jnp.float32),
    mesh=_mesh,
    compiler_params=pltpu.CompilerParams(use_tc_tiling_on_sc=False,
                                         needs_layout_passes=False),
    scratch_types=[
        pltpu.VMEM((2, _B), jnp.int32),
        pltpu.VMEM((2, _B), jnp.int32),
        pltpu.VMEM((2, _B // 2), jnp.int32),
        pltpu.VMEM((2, _B // 2), jnp.int32),
        pltpu.VMEM((_NAT * 16,), jnp.float32),
        pltpu.VMEM((_B, _D), jnp.bfloat16),
        pltpu.VMEM((_B, _D), jnp.bfloat16),
        pltpu.VMEM((_B // 2, _D), jnp.float32),
        pltpu.VMEM((_B // 2, _D), jnp.float32),
        pltpu.VMEM((_ZR, _D), jnp.float32),
        pltpu.VMEM_SHARED((_NPAD, _D), jnp.float32),
        pltpu.SemaphoreType.DMA,
        pltpu.SemaphoreType.DMA,
        pltpu.SemaphoreType.DMA,
        pltpu.SemaphoreType.DMA,
        pltpu.SemaphoreType.DMA,
        pltpu.SemaphoreType.DMA,
        pltpu.SemaphoreType.DMA,
        pltpu.SemaphoreType.DMA,
    ],
)
def _sc_agg(h_hbm, sr_hbm, dstr_hbm, alpha_hbm, out_hbm,
            pk0, pk1, dc0, dc1, alpha_v, rw0, rw1, sbA, sbB, zbuf_v, agg_sh,
            sp0, sp1, sd0, sd1, sr0, sr1, scA, scB):
    cid = lax.axis_index("c")
    sid = lax.axis_index("s")
    wid = cid * _NS + sid
    pks, dcs, rws = [pk0, pk1], [dc0, dc1], [rw0, rw1]
    sbs, scs = [sbA, sbB], [scA, scB]
    sps, sds, srs = [sp0, sp1], [sd0, sd1], [sr0, sr1]

    pltpu.sync_copy(alpha_hbm, alpha_v)

    zv = jnp.zeros((16,), jnp.float32)
    for r in range(_ZR):
        for c in range(_D // 16):
            zbuf_v[r, pl.ds(c * 16, 16)] = zv

    def _zcp(k, carry):
        pltpu.sync_copy(zbuf_v, agg_sh.at[pl.ds(sid * _RPS + k * _ZR, _ZR)])
        return carry

    lax.fori_loop(0, _RPS // _ZR, _zcp, 0)
    plsc.subcore_barrier()

    def _issue_pk(j, b):
        pltpu.async_copy(sr_hbm.at[wid, j], pks[b], sps[b])

    def _issue_dst(j, b):
        pltpu.async_copy(dstr_hbm.at[wid, j], dcs[b], sds[b])

    def _issue_gather(b):
        pltpu.async_copy(h_hbm.at[pks[b].at[0]], rws[b], srs[b])

    def _wait_pk(b):
        pltpu.make_async_copy(sr_hbm.at[wid, 0], pks[b], sps[b]).wait()

    def _wait_dst(b):
        pltpu.make_async_copy(dstr_hbm.at[wid, 0], dcs[b], sds[b]).wait()

    def _wait_gather(b):
        pltpu.make_async_copy(h_hbm.at[pks[b].at[0]], rws[b], srs[b]).wait()

    def _issue_scatter(b, h):
        pltpu.async_copy(sbs[h], agg_sh.at[dcs[b].at[h]], scs[h], add=True)

    def _wait_scatter(h):
        pltpu.make_async_copy(sbs[h], agg_sh.at[dcs[0].at[h]], scs[h]).wait()

    def _lookup(b):
        return [_alpha16(alpha_v, pks[b][1, pl.ds(eb * 16, 16)])
                for eb in range(_B // 16)]

    def _scale_half(b, a16s, h):
        for el in range(_B // 2):
            e = h * (_B // 2) + el
            ae = _lane_splat(a16s[e // 16], e % 16)
            for g in range(_D // 32):
                v = rws[b][e, pl.ds(g * 32, 32)]
                pa, pb = plsc.unpack(v, format=plsc.PackFormat.INTERLEAVED)
                sbs[h][el, pl.ds(g * 32, 16)] = pa * ae
                sbs[h][el, pl.ds(g * 32 + 16, 16)] = pb * ae

    def _sub_iter(j, b):
        b1 = 1 - b
        _wait_pk(b1)
        _issue_gather(b1)
        a16s = _lookup(b)
        _issue_pk(jnp.minimum(j + 2, _NCH - 1), b)
        _wait_gather(b)
        _wait_dst(b)
        _wait_scatter(0)
        _scale_half(b, a16s, 0)
        _issue_scatter(b, 0)
        _wait_scatter(1)
        _issue_dst(jnp.minimum(j + 1, _NCH - 1), b1)
        _scale_half(b, a16s, 1)
        _issue_scatter(b, 1)

    _issue_pk(0, 0)
    _issue_pk(1, 1)
    _issue_dst(0, 0)
    _issue_dst(1, 1)
    _wait_pk(0)
    _issue_gather(0)
    _wait_pk(1)
    _issue_gather(1)
    a16s0 = _lookup(0)
    _issue_pk(2, 0)
    _wait_gather(0)
    _wait_dst(0)
    _scale_half(0, a16s0, 0)
    _issue_scatter(0, 0)
    _scale_half(0, a16s0, 1)
    _issue_scatter(0, 1)

    def _pair(i, carry):
        _sub_iter(2 * i + 1, 1)
        _sub_iter(2 * i + 2, 0)
        return carry

    lax.fori_loop(0, (_NCH - 1) // 2, _pair, 0)

    _wait_scatter(0)
    _wait_scatter(1)
    _wait_pk(0)
    _wait_dst(1)
    _wait_gather(1)

    plsc.subcore_barrier()

    sl = pl.ds(sid * _RPS, _RPS)
    pltpu.sync_copy(agg_sh.at[sl], out_hbm.at[cid].at[sl])


def _tc_body(agg_ref, h_ref, wp_ref, w_ref, b_ref, g_ref, be_ref, out_ref):
    x = agg_ref[0, :_N] + agg_ref[1, :_N]
    y = (jnp.dot(x, wp_ref[...], preferred_element_type=jnp.float32)
         + jnp.dot(h_ref[...], w_ref[...], preferred_element_type=jnp.float32))
    y = y + b_ref[...]
    mu = jnp.mean(y, axis=0, keepdims=True)
    d = y - mu
    var = jnp.mean(d * d, axis=0, keepdims=True)
    out_ref[...] = jnp.tanh(d * lax.rsqrt(var + 1e-5) * g_ref[...] + be_ref[...])


_tc_layer = pl.pallas_call(
    _tc_body,
    out_shape=jax.ShapeDtypeStruct((_N, _D), jnp.float32),
)


_PERM = tuple(32 * g + 2 * k + p
              for g in range(_D // 32) for p in range(2) for k in range(16))
_PERM_ARR = _np.asarray(_PERM, dtype=_np.int32)


def kernel(entity_embed, edge, alpha0, W0, b0, gamma0, beta0,
           alpha1, W1, b1, gamma1, beta1):
    edge = edge.astype(jnp.int32)
    src = edge[:, 0].reshape(_NW, _NCH, 1, _B)
    rel = (edge[:, 1] % _NREL).reshape(_NW, _NCH, 1, _B)
    sr = jnp.concatenate([src, rel], axis=2)
    dst = edge[:, 2].reshape(_NW, _NCH, 2, _B // 2)
    apad = jnp.zeros((_NAT * 16 - _NREL,), jnp.float32)
    a0 = jnp.concatenate([alpha0, apad])
    a1 = jnp.concatenate([alpha1, apad])
    b0r, g0r, be0r = b0.reshape(1, _D), gamma0.reshape(1, _D), beta0.reshape(1, _D)
    b1r, g1r, be1r = b1.reshape(1, _D), gamma1.reshape(1, _D), beta1.reshape(1, _D)

    hb0 = entity_embed.astype(jnp.bfloat16)
    agg = _sc_agg(hb0, sr, dst, a0)
    h1 = _tc_layer(agg, entity_embed, W0[_PERM_ARR, :], W0, b0r, g0r, be0r)
    hb1 = h1.astype(jnp.bfloat16)
    agg2 = _sc_agg(hb1, sr, dst, a1)
    h2 = _tc_layer(agg2, h1, W1[_PERM_ARR, :], W1, b1r, g1r, be1r)
    return h2
